# Initial kernel scaffold; baseline (speedup 1.0000x reference)
import jax
import jax.numpy as jnp
from jax.experimental import pallas as pl


def kernel(coords, features):
    raise NotImplementedError("write your pallas kernel here")


# speedup vs baseline: 2.1289x; 2.1289x over previous
import functools
import jax
import jax.numpy as jnp
from jax import lax
from jax.experimental import pallas as pl
from jax.experimental.pallas import tpu as pltpu
from jax.experimental.pallas import tpu_sc as plsc

RES = 32
V = RES ** 3
B = 4
N = 65536
C = 64

NC = 2
NS = 16
CH = C // NC
PT = N // NS
VT = V // NS
CHUNK = 512
NCHUNK = PT // CHUNK
QPC = CHUNK // 128
ROWS = N // 128
RPT = ROWS // NS


def _coords_body(ct_ref, vc_ref, idx_ref):
    c3 = ct_ref[0]
    mean = jnp.mean(c3, axis=1, keepdims=True)
    cc = c3 - mean
    n2 = jnp.sum(cc * cc, axis=0, keepdims=True)
    m = jnp.sqrt(jnp.max(n2))
    cn = cc / (m * 2.0) + 0.5
    v = jnp.clip(cn * float(RES), 0.0, float(RES - 1))
    vc_ref[0] = v
    vi = jnp.round(v).astype(jnp.int32)
    idx_ref[0] = vi[0:1] * (RES * RES) + vi[1:2] * RES + vi[2:3]


def _coords_tc(ct):
    return pl.pallas_call(
        _coords_body,
        grid=(B,),
        in_specs=[pl.BlockSpec((1, 3, N), lambda b: (b, 0, 0))],
        out_specs=[
            pl.BlockSpec((1, 3, N), lambda b: (b, 0, 0)),
            pl.BlockSpec((1, 1, N), lambda b: (b, 0, 0)),
        ],
        out_shape=[
            jax.ShapeDtypeStruct((B, 3, N), jnp.float32),
            jax.ShapeDtypeStruct((B, 1, N), jnp.int32),
        ],
    )(ct)


_MESH = plsc.VectorSubcoreMesh(
    core_axis_name="c", subcore_axis_name="s", num_cores=NC, num_subcores=NS)


@functools.partial(
    pl.kernel,
    out_type=[
        jax.ShapeDtypeStruct((B, V, C), jnp.float32),
        jax.ShapeDtypeStruct((B, V), jnp.float32),
    ],
    mesh=_MESH,
    compiler_params=pltpu.CompilerParams(use_tc_tiling_on_sc=False),
    scratch_types=[
        pltpu.VMEM((CHUNK, CH), jnp.float32),
        pltpu.VMEM((RPT, 128), jnp.int32),
        pltpu.VMEM((128,), jnp.float32),
        pltpu.VMEM((CHUNK, CH), jnp.float32),
        pltpu.VMEM((VT,), jnp.float32),
        pltpu.VMEM_SHARED((V, CH), jnp.float32),
        pltpu.VMEM_SHARED((V,), jnp.float32),
    ],
)
def _scatter_sc(feats, idxs, ones_h, z2d, z1d, sums_out, cnts_out,
                fv, iv, ones_v, zv, zc, sums_sp, cnts_sp):
    c = lax.axis_index("c")
    s = lax.axis_index("s")
    ch0 = c * CH
    v0 = s * VT
    pltpu.sync_copy(ones_h, ones_v)
    pltpu.sync_copy(z2d, zv)
    pltpu.sync_copy(z1d, zc)
    for b in range(B):
        for z in range(VT // CHUNK):
            pltpu.sync_copy(zv, sums_sp.at[pl.ds(v0 + z * CHUNK, CHUNK), :])
        pltpu.sync_copy(zc, cnts_sp.at[pl.ds(v0, VT)])
        plsc.subcore_barrier()

        pltpu.sync_copy(idxs.at[b, pl.ds(s * RPT, RPT), :], iv)

        @pl.loop(0, NCHUNK)
        def _chunk(j):
            p0 = s * PT + j * CHUNK
            pltpu.sync_copy(feats.at[b, pl.ds(p0, CHUNK), pl.ds(ch0, CH)], fv)
            for q in range(QPC):
                row = iv.at[j * QPC + q]
                pltpu.sync_copy(fv.at[pl.ds(q * 128, 128), :],
                                sums_sp.at[row], add=True)
                pltpu.sync_copy(ones_v, cnts_sp.at[row], add=True)

        plsc.subcore_barrier()

        pltpu.sync_copy(sums_sp.at[pl.ds(v0, VT), :],
                        sums_out.at[b, pl.ds(v0, VT), pl.ds(ch0, CH)])

        @pl.when(c == 0)
        def _():
            pltpu.sync_copy(cnts_sp.at[pl.ds(v0, VT)],
                            cnts_out.at[b, pl.ds(v0, VT)])


def _div_body(s_ref, c_ref, o_ref):
    cnt = jnp.maximum(c_ref[0], 1.0)
    o_ref[0] = s_ref[0] / cnt


def _div_tc(sums, cnts3):
    return pl.pallas_call(
        _div_body,
        grid=(B, V // VT),
        in_specs=[
            pl.BlockSpec((1, VT, C), lambda b, i: (b, i, 0)),
            pl.BlockSpec((1, VT, 1), lambda b, i: (b, i, 0)),
        ],
        out_specs=pl.BlockSpec((1, VT, C), lambda b, i: (b, i, 0)),
        out_shape=jax.ShapeDtypeStruct((B, V, C), jnp.float32),
    )(sums, cnts3)


def kernel(coords, features):
    ct = coords.transpose(0, 2, 1)
    vc_t, idx = _coords_tc(ct)
    voxel_coords = vc_t.transpose(0, 2, 1)
    idx_rows = idx.reshape(B, ROWS, 128)
    ones_h = jnp.ones((128,), jnp.float32)
    z2d = jnp.zeros((CHUNK, CH), jnp.float32)
    z1d = jnp.zeros((VT,), jnp.float32)
    sums, cnts = _scatter_sc(features, idx_rows, ones_h, z2d, z1d)
    vox = _div_tc(sums, cnts.reshape(B, V, 1))
    voxel_features = vox.reshape(B, RES, RES, RES, C)
    return voxel_coords, voxel_features
